# Initial kernel scaffold; baseline (speedup 1.0000x reference)
import jax
import jax.numpy as jnp
from jax.experimental import pallas as pl


def kernel(x_list, edge_index, W, b):
    raise NotImplementedError("write your pallas kernel here")


# speedup vs baseline: 15.0637x; 15.0637x over previous
import functools

import jax
import jax.numpy as jnp
from jax import lax
from jax.experimental import pallas as pl
from jax.experimental.pallas import tpu as pltpu
from jax.experimental.pallas import tpu_sc as plsc

_LANES = 16
_NW = 32


def _proj_body(x_ref, w_ref, o_ref):
    o_ref[...] = jnp.dot(x_ref[...], w_ref[...],
                         preferred_element_type=jnp.float32,
                         precision=lax.Precision.HIGHEST)


def _node_projections(x, wm):
    n = x.shape[0]
    return pl.pallas_call(
        _proj_body,
        out_shape=jax.ShapeDtypeStruct((n, 128), jnp.float32),
    )(x, wm)


def _make_sc_gather(n_nodes, e_pad):
    per_w = e_pad // _NW
    mesh = plsc.VectorSubcoreMesh(core_axis_name="c", subcore_axis_name="s")

    @functools.partial(
        pl.kernel,
        mesh=mesh,
        out_type=jax.ShapeDtypeStruct((e_pad,), jnp.float32),
        compiler_params=pltpu.CompilerParams(needs_layout_passes=False),
        scratch_types=[
            pltpu.VMEM((per_w,), jnp.int32),
            pltpu.VMEM((per_w,), jnp.int32),
            pltpu.VMEM((n_nodes,), jnp.float32),
            pltpu.VMEM((n_nodes,), jnp.float32),
            pltpu.VMEM((per_w,), jnp.float32),
        ],
    )
    def sc_gather(u_hbm, v_hbm, s_hbm, t_hbm, out_hbm, u_v, v_v, s_v, t_v, o_v):
        wid = lax.axis_index("s") * 2 + lax.axis_index("c")
        base = wid * per_w
        pltpu.sync_copy(u_hbm.at[pl.ds(base, per_w)], u_v)
        pltpu.sync_copy(v_hbm.at[pl.ds(base, per_w)], v_v)
        pltpu.sync_copy(s_hbm, s_v)
        pltpu.sync_copy(t_hbm, t_v)

        def body(i, carry):
            off = i * _LANES
            u = u_v[pl.ds(off, _LANES)]
            v = v_v[pl.ds(off, _LANES)]
            sv = plsc.load_gather(s_v, [u])
            tv = plsc.load_gather(t_v, [v])
            o_v[pl.ds(off, _LANES)] = sv + tv
            return carry

        lax.fori_loop(0, per_w // _LANES, body, 0)
        pltpu.sync_copy(o_v, out_hbm.at[pl.ds(base, per_w)])

    return sc_gather


def _make_norm(n_real_rows, n_real):
    def norm_body(l_ref, o_ref):
        l = l_ref[...]
        rows = lax.broadcasted_iota(jnp.int32, l.shape, 0)
        lm = jnp.where(rows < n_real_rows, l, 0.0)
        s = jnp.sum(lm)
        ss = jnp.sum(lm * lm)
        n = jnp.float32(n_real)
        mean = s / n
        var = (ss - s * s / n) / (n - 1.0)
        inv = lax.rsqrt(var)
        o_ref[...] = jax.nn.sigmoid((l - mean) * inv)

    return norm_body


def kernel(x_list, edge_index, W, b):
    del b
    n_nodes, d = x_list.shape
    e = edge_index.shape[1]

    wm = jnp.zeros((d, 128), jnp.float32).at[:, :2].set(
        W[0].reshape(2, d).T)
    st = _node_projections(x_list, wm)
    s = st[:, 0]
    t = st[:, 1]

    e_pad = ((e + _NW * _LANES - 1) // (_NW * _LANES)) * (_NW * _LANES)
    idx = edge_index.astype(jnp.int32)
    pad = e_pad - e
    u_idx = jnp.pad(idx[0], (0, pad))
    v_idx = jnp.pad(idx[1], (0, pad))

    logits = _make_sc_gather(n_nodes, e_pad)(u_idx, v_idx, s, t)

    n_cols = 128
    n_rows_pad = e_pad // n_cols
    n_real_rows = e // n_cols
    out = pl.pallas_call(
        _make_norm(n_real_rows, e),
        out_shape=jax.ShapeDtypeStruct((n_rows_pad, n_cols), jnp.float32),
    )(logits.reshape(n_rows_pad, n_cols))
    return out.reshape(-1)[:e]


# speedup vs baseline: 16.0837x; 1.0677x over previous
import functools

import jax
import jax.numpy as jnp
from jax import lax
from jax.experimental import pallas as pl
from jax.experimental.pallas import tpu as pltpu
from jax.experimental.pallas import tpu_sc as plsc

_LANES = 16
_NW = 32


def _proj_body(x_ref, w_ref, o_ref):
    o_ref[...] = jnp.dot(x_ref[...], w_ref[...],
                         preferred_element_type=jnp.float32,
                         precision=lax.Precision.HIGHEST)


def _node_projections(x, wm):
    n = x.shape[0]
    return pl.pallas_call(
        _proj_body,
        out_shape=jax.ShapeDtypeStruct((n, wm.shape[1]), jnp.float32),
    )(x, wm)


def _make_sc_gather(n_nodes, n_edges):
    per_w = n_edges // _NW
    full_vregs = per_w // _LANES
    tail = per_w - full_vregs * _LANES
    per_w_pad = (full_vregs + (1 if tail else 0)) * _LANES
    mesh = plsc.VectorSubcoreMesh(core_axis_name="c", subcore_axis_name="s")

    @functools.partial(
        pl.kernel,
        mesh=mesh,
        out_type=jax.ShapeDtypeStruct((n_edges,), jnp.float32),
        compiler_params=pltpu.CompilerParams(needs_layout_passes=False),
        scratch_types=[
            pltpu.VMEM((per_w_pad,), jnp.int32),
            pltpu.VMEM((per_w_pad,), jnp.int32),
            pltpu.VMEM((n_nodes,), jnp.float32),
            pltpu.VMEM((n_nodes,), jnp.float32),
            pltpu.VMEM((per_w_pad,), jnp.float32),
        ],
    )
    def sc_gather(u_hbm, v_hbm, s_hbm, t_hbm, out_hbm, u_v, v_v, s_v, t_v, o_v):
        wid = lax.axis_index("s") * 2 + lax.axis_index("c")
        base = wid * per_w
        pltpu.sync_copy(u_hbm.at[pl.ds(base, per_w)], u_v.at[pl.ds(0, per_w)])
        pltpu.sync_copy(v_hbm.at[pl.ds(base, per_w)], v_v.at[pl.ds(0, per_w)])
        pltpu.sync_copy(s_hbm, s_v)
        pltpu.sync_copy(t_hbm, t_v)

        @plsc.parallel_loop(0, full_vregs * _LANES, _LANES, unroll=8)
        def _(off):
            u = u_v[pl.ds(off, _LANES)]
            v = v_v[pl.ds(off, _LANES)]
            sv = plsc.load_gather(s_v, [u])
            tv = plsc.load_gather(t_v, [v])
            o_v[pl.ds(off, _LANES)] = sv + tv

        if tail:
            off = full_vregs * _LANES
            mask = lax.iota(jnp.int32, _LANES) < tail
            u = jnp.where(mask, u_v[pl.ds(off, _LANES)], 0)
            v = jnp.where(mask, v_v[pl.ds(off, _LANES)], 0)
            sv = plsc.load_gather(s_v, [u], mask=mask)
            tv = plsc.load_gather(t_v, [v], mask=mask)
            o_v[pl.ds(off, _LANES)] = sv + tv

        pltpu.sync_copy(o_v.at[pl.ds(0, per_w)], out_hbm.at[pl.ds(base, per_w)])

    return sc_gather


def _norm_body(l_ref, o_ref):
    l = l_ref[...]
    s = jnp.sum(l)
    ss = jnp.sum(l * l)
    n = jnp.float32(l.size)
    mean = s / n
    var = (ss - s * s / n) / (n - 1.0)
    inv = lax.rsqrt(var)
    o_ref[...] = jax.nn.sigmoid((l - mean) * inv)


def kernel(x_list, edge_index, W, b):
    del b
    n_nodes, d = x_list.shape
    e = edge_index.shape[1]

    wm = jnp.zeros((d, 8), jnp.float32).at[:, :2].set(W[0].reshape(2, d).T)
    st = _node_projections(x_list, wm)
    s = st[:, 0]
    t = st[:, 1]

    idx = edge_index.astype(jnp.int32)
    logits = _make_sc_gather(n_nodes, e)(idx[0], idx[1], s, t)

    n_cols = 128
    n_rows = e // n_cols
    out = pl.pallas_call(
        _norm_body,
        out_shape=jax.ShapeDtypeStruct((n_rows, n_cols), jnp.float32),
    )(logits.reshape(n_rows, n_cols))
    return out.reshape(-1)


# speedup vs baseline: 19.2144x; 1.1947x over previous
import functools

import jax
import jax.numpy as jnp
from jax import lax
from jax.experimental import pallas as pl
from jax.experimental.pallas import tpu as pltpu
from jax.experimental.pallas import tpu_sc as plsc

_LANES = 16
_NW = 32


def _proj_body(x_ref, w_ref, o_ref):
    o_ref[...] = lax.dot_general(
        w_ref[...], x_ref[...],
        dimension_numbers=(((1,), (1,)), ((), ())),
        preferred_element_type=jnp.float32,
        precision=lax.Precision.HIGHEST)


def _node_projections(x, wm):
    n = x.shape[0]
    return pl.pallas_call(
        _proj_body,
        out_shape=jax.ShapeDtypeStruct((wm.shape[0], n), jnp.float32),
    )(x, wm)


def _make_sc_gather(n_nodes, n_edges):
    per_w = n_edges // _NW
    full_vregs = per_w // _LANES
    tail = per_w - full_vregs * _LANES
    per_w_pad = (full_vregs + (1 if tail else 0)) * _LANES
    mesh = plsc.VectorSubcoreMesh(core_axis_name="c", subcore_axis_name="s")

    @functools.partial(
        pl.kernel,
        mesh=mesh,
        out_type=jax.ShapeDtypeStruct((n_edges,), jnp.float32),
        compiler_params=pltpu.CompilerParams(needs_layout_passes=False),
        scratch_types=[
            pltpu.VMEM((per_w_pad,), jnp.int32),
            pltpu.VMEM((per_w_pad,), jnp.int32),
            pltpu.VMEM((n_nodes,), jnp.float32),
            pltpu.VMEM((n_nodes,), jnp.float32),
            pltpu.VMEM((per_w_pad,), jnp.float32),
        ],
    )
    def sc_gather(u_hbm, v_hbm, s_hbm, t_hbm, out_hbm, u_v, v_v, s_v, t_v, o_v):
        wid = lax.axis_index("s") * 2 + lax.axis_index("c")
        base = wid * per_w
        pltpu.sync_copy(u_hbm.at[pl.ds(base, per_w)], u_v.at[pl.ds(0, per_w)])
        pltpu.sync_copy(v_hbm.at[pl.ds(base, per_w)], v_v.at[pl.ds(0, per_w)])
        pltpu.sync_copy(s_hbm, s_v)
        pltpu.sync_copy(t_hbm, t_v)

        @plsc.parallel_loop(0, full_vregs * _LANES, _LANES, unroll=8)
        def _(off):
            u = u_v[pl.ds(off, _LANES)]
            v = v_v[pl.ds(off, _LANES)]
            sv = plsc.load_gather(s_v, [u])
            tv = plsc.load_gather(t_v, [v])
            o_v[pl.ds(off, _LANES)] = sv + tv

        if tail:
            off = full_vregs * _LANES
            mask = lax.iota(jnp.int32, _LANES) < tail
            u = jnp.where(mask, u_v[pl.ds(off, _LANES)], 0)
            v = jnp.where(mask, v_v[pl.ds(off, _LANES)], 0)
            sv = plsc.load_gather(s_v, [u], mask=mask)
            tv = plsc.load_gather(t_v, [v], mask=mask)
            o_v[pl.ds(off, _LANES)] = sv + tv

        pltpu.sync_copy(o_v.at[pl.ds(0, per_w)], out_hbm.at[pl.ds(base, per_w)])

    return sc_gather


def _norm_body(l_ref, o_ref):
    l = l_ref[...]
    s = jnp.sum(l)
    ss = jnp.sum(l * l)
    n = jnp.float32(l.size)
    mean = s / n
    var = (ss - s * s / n) / (n - 1.0)
    inv = lax.rsqrt(var)
    o_ref[...] = jax.nn.sigmoid((l - mean) * inv)


def kernel(x_list, edge_index, W, b):
    del b
    n_nodes, d = x_list.shape
    e = edge_index.shape[1]

    wm = jnp.zeros((8, d), jnp.float32).at[:2, :].set(W[0].reshape(2, d))
    st = _node_projections(x_list, wm)
    s = st[0]
    t = st[1]

    idx = edge_index.astype(jnp.int32)
    logits = _make_sc_gather(n_nodes, e)(idx[0], idx[1], s, t)

    n_cols = 128
    n_rows = e // n_cols
    out = pl.pallas_call(
        _norm_body,
        out_shape=jax.ShapeDtypeStruct((n_rows, n_cols), jnp.float32),
    )(logits.reshape(n_rows, n_cols))
    return out.reshape(-1)


# speedup vs baseline: 29.4560x; 1.5330x over previous
import functools

import jax
import jax.numpy as jnp
from jax import lax
from jax.experimental import pallas as pl
from jax.experimental.pallas import tpu as pltpu
from jax.experimental.pallas import tpu_sc as plsc

_LANES = 16
_NW = 32


def _make_proj(n, d, blk):
    del blk
    def body(x_ref, w_ref, o_ref):
        w2 = jnp.concatenate([w_ref[:, :d], w_ref[:, d:]], axis=0)
        o_ref[...] = lax.dot_general(
            w2, x_ref[...],
            dimension_numbers=(((1,), (1,)), ((), ())),
            preferred_element_type=jnp.float32,
            precision=lax.Precision.DEFAULT)

    return pl.pallas_call(
        body,
        out_shape=jax.ShapeDtypeStruct((2, n), jnp.float32),
    )


def _make_sc_gather(n_nodes, n_edges):
    nblk = n_edges // 128
    main_blk = nblk // _NW
    per_w = main_blk * 128
    rem = nblk - main_blk * _NW
    rem_base = _NW * per_w
    mesh = plsc.VectorSubcoreMesh(core_axis_name="c", subcore_axis_name="s")

    @functools.partial(
        pl.kernel,
        mesh=mesh,
        out_type=jax.ShapeDtypeStruct((n_edges,), jnp.float32),
        compiler_params=pltpu.CompilerParams(needs_layout_passes=False),
        scratch_types=[
            pltpu.VMEM((2, per_w), jnp.int32),
            pltpu.VMEM((2, 128), jnp.int32),
            pltpu.VMEM((n_nodes,), jnp.float32),
            pltpu.VMEM((n_nodes,), jnp.float32),
            pltpu.VMEM((per_w,), jnp.float32),
            pltpu.VMEM((128,), jnp.float32),
            pltpu.SemaphoreType.DMA,
        ],
    )
    def sc_gather(idx_hbm, s_hbm, t_hbm, out_hbm,
                  uv_v, uv2_v, s_v, t_v, o_v, o2_v, sem):
        wid = lax.axis_index("s") * 2 + lax.axis_index("c")
        base = wid * per_w
        cps = [
            pltpu.async_copy(idx_hbm.at[:, pl.ds(base, per_w)], uv_v, sem),
            pltpu.async_copy(s_hbm, s_v, sem),
            pltpu.async_copy(t_hbm, t_v, sem),
        ]
        for cp in cps:
            cp.wait()

        @plsc.parallel_loop(0, per_w, _LANES, unroll=8)
        def _(off):
            u = uv_v[0, pl.ds(off, _LANES)]
            v = uv_v[1, pl.ds(off, _LANES)]
            sv = plsc.load_gather(s_v, [u])
            tv = plsc.load_gather(t_v, [v])
            o_v[pl.ds(off, _LANES)] = sv + tv

        pltpu.sync_copy(o_v, out_hbm.at[pl.ds(base, per_w)])

        @pl.when(wid < rem)
        def _():
            base2 = rem_base + wid * 128
            pltpu.sync_copy(idx_hbm.at[:, pl.ds(base2, 128)], uv2_v)

            @plsc.parallel_loop(0, 128, _LANES, unroll=8)
            def _(off):
                u = uv2_v[0, pl.ds(off, _LANES)]
                v = uv2_v[1, pl.ds(off, _LANES)]
                sv = plsc.load_gather(s_v, [u])
                tv = plsc.load_gather(t_v, [v])
                o2_v[pl.ds(off, _LANES)] = sv + tv

            pltpu.sync_copy(o2_v, out_hbm.at[pl.ds(base2, 128)])

    return sc_gather


def _norm_body(l_ref, o_ref):
    l = l_ref[...]
    s = jnp.sum(l)
    ss = jnp.sum(l * l)
    n = jnp.float32(l.size)
    mean = s / n
    var = (ss - s * s / n) / (n - 1.0)
    inv = lax.rsqrt(var)
    o_ref[...] = jax.nn.sigmoid((l - mean) * inv)


def kernel(x_list, edge_index, W, b):
    del b
    n_nodes, d = x_list.shape
    e = edge_index.shape[1]

    st = _make_proj(n_nodes, d, 1000)(x_list, W)
    s = st[0]
    t = st[1]

    idx = edge_index.astype(jnp.int32)
    logits = _make_sc_gather(n_nodes, e)(idx, s, t)

    n_cols = 128
    n_rows = e // n_cols
    out = pl.pallas_call(
        _norm_body,
        out_shape=jax.ShapeDtypeStruct((n_rows, n_cols), jnp.float32),
    )(logits.reshape(n_rows, n_cols))
    return out.reshape(-1)
